# speedup vs baseline: 1.2468x; 1.2468x over previous
import functools

import jax
import jax.numpy as jnp
from jax import lax
from jax.experimental import pallas as pl
from jax.experimental.pallas import tpu as pltpu
from jax.experimental.pallas import tpu_sc as plsc

_NUM_CORES = 2
_NUM_SUBCORES = 16
_NW = _NUM_CORES * _NUM_SUBCORES
_CHUNK = 8


def kernel(input_ids, embed_table):
    batch, seq = input_ids.shape
    _, d_model = embed_table.shape
    n = batch * seq
    b_per_w = n // _NW
    nchunks = b_per_w // _CHUNK

    ids_flat = input_ids.reshape(n).astype(jnp.int32)

    mesh = plsc.VectorSubcoreMesh(
        core_axis_name="c", subcore_axis_name="s",
        num_cores=_NUM_CORES, num_subcores=_NUM_SUBCORES)

    @functools.partial(
        pl.kernel,
        out_type=jax.ShapeDtypeStruct((n, d_model), jnp.float32),
        mesh=mesh,
        scratch_types=[
            pltpu.VMEM((b_per_w,), jnp.int32),
            pltpu.VMEM((_CHUNK, d_model), jnp.float32),
            pltpu.SemaphoreType.DMA,
        ],
    )
    def run(table_hbm, ids_hbm, out_hbm, idx_v, rows_v, sem):
        wid = lax.axis_index("s") * _NUM_CORES + lax.axis_index("c")
        base = wid * b_per_w
        pltpu.sync_copy(ids_hbm.at[pl.ds(base, b_per_w)], idx_v)

        @pl.loop(0, nchunks)
        def _(i):
            off = i * _CHUNK
            pltpu.async_copy(
                table_hbm.at[idx_v.at[pl.ds(off, _CHUNK)]], rows_v, sem
            ).wait()
            pltpu.sync_copy(rows_v, out_hbm.at[pl.ds(base + off, _CHUNK)])

    out = run(embed_table, ids_flat)
    return out.reshape(batch, seq, d_model)


# speedup vs baseline: 1.7732x; 1.4222x over previous
import functools

import jax
import jax.numpy as jnp
from jax import lax
from jax.experimental import pallas as pl
from jax.experimental.pallas import tpu as pltpu
from jax.experimental.pallas import tpu_sc as plsc

_NUM_CORES = 2
_NUM_SUBCORES = 16
_NW = _NUM_CORES * _NUM_SUBCORES
_CHUNK = 8
_NBUF = 4


def kernel(input_ids, embed_table):
    batch, seq = input_ids.shape
    _, d_model = embed_table.shape
    n = batch * seq
    b_per_w = n // _NW
    nchunks = b_per_w // _CHUNK

    ids_flat = input_ids.reshape(n).astype(jnp.int32)

    mesh = plsc.VectorSubcoreMesh(
        core_axis_name="c", subcore_axis_name="s",
        num_cores=_NUM_CORES, num_subcores=_NUM_SUBCORES)

    scratch = [pltpu.VMEM((b_per_w,), jnp.int32)]
    scratch += [pltpu.VMEM((_CHUNK, d_model), jnp.float32)] * _NBUF
    scratch += [pltpu.SemaphoreType.DMA] * (2 * _NBUF)

    @functools.partial(
        pl.kernel,
        out_type=jax.ShapeDtypeStruct((n, d_model), jnp.float32),
        mesh=mesh,
        scratch_types=scratch,
    )
    def run(table_hbm, ids_hbm, out_hbm, idx_v, *rest):
        bufs = rest[:_NBUF]
        gsems = rest[_NBUF:2 * _NBUF]
        osems = rest[2 * _NBUF:]

        wid = lax.axis_index("s") * _NUM_CORES + lax.axis_index("c")
        base = wid * b_per_w
        pltpu.sync_copy(ids_hbm.at[pl.ds(base, b_per_w)], idx_v)

        def start_gather(i, b):
            pltpu.async_copy(
                table_hbm.at[idx_v.at[pl.ds(i * _CHUNK, _CHUNK)]],
                bufs[b], gsems[b])

        def wait_gather(i, b):
            pltpu.make_async_copy(
                table_hbm.at[idx_v.at[pl.ds(i * _CHUNK, _CHUNK)]],
                bufs[b], gsems[b]).wait()

        def start_out(i, b):
            pltpu.async_copy(
                bufs[b], out_hbm.at[pl.ds(base + i * _CHUNK, _CHUNK)],
                osems[b])

        def wait_out(i, b):
            pltpu.make_async_copy(
                bufs[b], out_hbm.at[pl.ds(base + i * _CHUNK, _CHUNK)],
                osems[b]).wait()

        start_gather(0, 0)
        start_gather(1, 1)
        wait_gather(0, 0); start_out(0, 0); start_gather(2, 2)
        wait_gather(1, 1); start_out(1, 1); start_gather(3, 3)
        wait_gather(2, 2); start_out(2, 2); wait_out(0, 0); start_gather(4, 0)
        wait_gather(3, 3); start_out(3, 3); wait_out(1, 1); start_gather(5, 1)

        @pl.loop(_NBUF, nchunks - _NBUF, step=_NBUF)
        def _(g):
            for b in range(_NBUF):
                j = g + b
                wait_gather(j, b)
                start_out(j, b)
                wait_out(j - 2, (b + 2) % _NBUF)
                start_gather(j + 2, (b + 2) % _NBUF)

        t = nchunks - _NBUF
        wait_gather(t, 0); start_out(t, 0)
        wait_out(t - 2, 2); start_gather(t + 2, 2)
        wait_gather(t + 1, 1); start_out(t + 1, 1)
        wait_out(t - 1, 3); start_gather(t + 3, 3)
        wait_gather(t + 2, 2); start_out(t + 2, 2)
        wait_gather(t + 3, 3); start_out(t + 3, 3)
        wait_out(t, 0); wait_out(t + 1, 1)
        wait_out(t + 2, 2); wait_out(t + 3, 3)

    out = run(embed_table, ids_flat)
    return out.reshape(batch, seq, d_model)
